# baseline (speedup 1.0000x reference)
import jax
import jax.numpy as jnp
from jax import lax
from jax.experimental import pallas as pl
from jax.experimental.pallas import tpu as pltpu
from jax.experimental.pallas import tpu_sc as plsc

_V = 2
_N = 8192
_FT = 128
_H = 128
_D = 64
_NEI = 10
_SAMP = 5
_ALPHA = 0.5
_BETA = 1.0
_SLOTS = _SAMP + 1

_NW = 32
_RPW = (_V * _N) // _NW


_CHUNK = 128
_NCH = _RPW // _CHUNK


def _sc_nei_body(x_hbm, sel_hbm, out_hbm, idx_v, rows_v, sem):
    wid = lax.axis_index("s") * 2 + lax.axis_index("c")
    base = wid * _RPW
    view = base // _N
    c0 = (wid % 16) * _NCH
    pltpu.sync_copy(sel_hbm.at[view, :, pl.ds(c0, _NCH), :], idx_v)
    first = [
        pltpu.async_copy(x_hbm.at[idx_v.at[0, c]],
                         rows_v.at[pl.ds(c * _CHUNK, _CHUNK), :], sem)
        for c in range(_NCH)
    ]
    for d in first:
        d.wait()
    adds = [
        pltpu.async_copy(x_hbm.at[idx_v.at[j, c]],
                         rows_v.at[pl.ds(c * _CHUNK, _CHUNK), :], sem, add=True)
        for j in range(1, _SLOTS)
        for c in range(_NCH)
    ]
    for d in adds:
        d.wait()
    pltpu.sync_copy(rows_v, out_hbm.at[pl.ds(base, _RPW)])


def _nei_sum_sc(x, sel):
    k = pl.kernel(
        _sc_nei_body,
        out_type=jax.ShapeDtypeStruct((_V * _N, _FT), jnp.float32),
        mesh=plsc.VectorSubcoreMesh(core_axis_name="c", subcore_axis_name="s"),
        scratch_types=[
            pltpu.VMEM((_SLOTS, _NCH, _CHUNK), jnp.int32),
            pltpu.VMEM((_RPW, _FT), jnp.float32),
            pltpu.SemaphoreType.DMA,
        ],
    )
    return k(x, sel.reshape(_V, _SLOTS, _N // _CHUNK, _CHUNK)).reshape(_V, _N, _FT)


_BM = 512
_NB = _N // _BM
_BM2 = 1024
_ND = _N // _BM2
_S0 = _NB
_S1 = _S0 + 3 * (_NB // 2)
_NT = _S1 + _ND


def _mega_body(adj_ref, x_ref, ew1_ref, eb1_ref, ew2_ref, eb2_ref, w0_ref,
               g_ref, bb_ref, nei_ref, dw1_ref, db1_ref, dw2_ref, db2_ref,
               out_ref, ha_v, hp_v, gram_v, ic_v, sum_v, a_v, be_v,
               misc_s, e1_s, e2_s):
    g = pl.program_id(0)
    eye = (lax.broadcasted_iota(jnp.int32, (_D, _D), 0)
           == lax.broadcasted_iota(jnp.int32, (_D, _D), 1)).astype(jnp.float32)
    r = g - _S0
    q = r // 3
    m = r % 3
    in_mid = (g >= _S0) & (g < _S1)
    is_dec = (g >= _S1) | (in_mid & (m == 2))

    @pl.when(jnp.logical_not(is_dec))
    def _spmm_phase():
        i = jnp.where(g < _S0, 0, 1)
        b = jnp.where(g < _S0, g, 2 * q + m)

        @pl.when(b == 0)
        def _():
            h = lax.dot_general(x_ref[...], ew1_ref[i], (((1,), (1,)), ((), ())),
                                preferred_element_type=jnp.float32)
            h = jnp.maximum(h + eb1_ref[i, 0:1, :], 0.0)
            ha = lax.dot_general(h, ew2_ref[i], (((1,), (1,)), ((), ())),
                                 preferred_element_type=jnp.float32)
            ha_v[i] = ha + eb2_ref[i, 0:1, :]
            gram_v[i] = jnp.zeros((_D, _D), jnp.float32)
            sum_v[i] = jnp.zeros((1, _D), jnp.float32)

            @pl.when(i == 1)
            def _():
                ic_v[...] = jnp.zeros((_D, _D), jnp.float32)
                e1_s[0, 0] = 0.0
                e2_s[0, 0] = 0.0

        hp = lax.dot_general(adj_ref[0], ha_v[i], (((1,), (0,)), ((), ())),
                             preferred_element_type=jnp.float32)
        hp_v[i, pl.ds(b * _BM, _BM)] = hp
        gram_v[i] += lax.dot_general(hp, hp, (((0,), (0,)), ((), ())),
                                     preferred_element_type=jnp.float32)
        sum_v[i] += jnp.sum(hp, axis=0, keepdims=True)

        @pl.when(i == 1)
        def _():
            ic_v[...] += lax.dot_general(hp, hp_v[0, pl.ds(b * _BM, _BM)],
                                         (((0,), (0,)), ((), ())),
                                         preferred_element_type=jnp.float32)

        @pl.when(b == _NB - 1)
        def _():
            gram = gram_v[i]
            rn = jnp.sqrt(jnp.sum(gram * gram, axis=1, keepdims=True))
            c = gram / jnp.maximum(rn, 1e-12)
            d = jnp.sum(c * eye, axis=1, keepdims=True)
            intra = jnp.sum((d - 1.0) ** 2) + (jnp.sum(c * c) - jnp.sum(d * d))

            @pl.when(i == 0)
            def _():
                misc_s[0, 0] = _ALPHA * intra

            s = sum_v[i] * (1.0 / _N)
            w0 = w0_ref[i]
            mean_u = lax.dot_general(s, w0, (((1,), (1,)), ((), ())),
                                     preferred_element_type=jnp.float32)
            m1 = lax.dot_general(w0, gram, (((1,), (0,)), ((), ())),
                                 preferred_element_type=jnp.float32)
            m2 = lax.dot_general(m1, w0, (((1,), (1,)), ((), ())),
                                 preferred_element_type=jnp.float32)
            ediag = jnp.sum(m2 * eye, axis=0, keepdims=True) * (1.0 / _N)
            var = ediag - mean_u * mean_u
            inv_std = lax.rsqrt(var + 1e-5)
            a = g_ref[i, 0:1, :] * inv_std
            a_v[i] = a
            be_v[i] = bb_ref[i, 0:1, :] - mean_u * a

            @pl.when(i == 1)
            def _():
                ic = ic_v[...]
                rn2 = jnp.sqrt(jnp.sum(ic * ic, axis=1, keepdims=True))
                icn = ic / jnp.maximum(rn2, 1e-12)
                misc_s[0, 0] += _ALPHA * intra - jnp.sum(icn * eye)

    @pl.when(is_dec)
    def _decoder_phase():
        i = jnp.where(g < _S1, 0, 1)
        bb = jnp.where(g < _S1, q, g - _S1)

        hp = hp_v[i, pl.ds(bb * _BM2, _BM2)]
        u = lax.dot_general(hp, w0_ref[i], (((1,), (1,)), ((), ())),
                            preferred_element_type=jnp.float32)
        t = jnp.maximum(u * a_v[i] + be_v[i], 0.0)
        t2 = lax.dot_general(t, dw1_ref[i], (((1,), (1,)), ((), ())),
                             preferred_element_type=jnp.float32) + db1_ref[i]
        r = lax.dot_general(jnp.maximum(t2, 0.0), dw2_ref[i],
                            (((1,), (1,)), ((), ())),
                            preferred_element_type=jnp.float32) + db2_ref[i]
        d1 = r - x_ref[pl.ds(bb * _BM2, _BM2)]
        d2 = r - nei_ref[0] * (1.0 / _SAMP)
        e1_s[0, 0] += jnp.sum(d1 * d1)
        e2_s[0, 0] += jnp.sum(d2 * d2)

        @pl.when(g == _NT - 1)
        def _():
            out_ref[0, 0] = (misc_s[0, 0]
                             + _BETA * ((e1_s[0, 0] + e2_s[0, 0]) * (1.0 / _N)))


def _mega(adj, x, ew1, eb1, ew2, eb2, w0, g, bb, nei_sum, dw1, db1, dw2, db2):
    def adj_idx(gi):
        r = gi - _S0
        q = r // 3
        m = r % 3
        v = jnp.where(gi < _S0, 0, 1)
        b_mid = jnp.where(m < 2, 2 * q + m, jnp.minimum(2 * q + 2, _NB - 1))
        b = jnp.where(gi < _S0, gi, jnp.where(gi < _S1, b_mid, _NB - 1))
        return (v, b, 0)

    def nei_idx(gi):
        q = (gi - _S0) // 3
        v = jnp.where(gi < _S1, 0, 1)
        bb_ = jnp.where(gi < _S0, 0, jnp.where(gi < _S1, q, gi - _S1))
        return (v, bb_, 0)

    return pl.pallas_call(
        _mega_body,
        grid=(_NT,),
        in_specs=[
            pl.BlockSpec((1, _BM, _N), adj_idx),
            pl.BlockSpec((_N, _FT), lambda gi: (0, 0)),
            pl.BlockSpec((_V, _H, _FT), lambda gi: (0, 0, 0)),
            pl.BlockSpec((_V, 1, _H), lambda gi: (0, 0, 0)),
            pl.BlockSpec((_V, _D, _H), lambda gi: (0, 0, 0)),
            pl.BlockSpec((_V, 1, _D), lambda gi: (0, 0, 0)),
            pl.BlockSpec((_V, _D, _D), lambda gi: (0, 0, 0)),
            pl.BlockSpec((_V, 1, _D), lambda gi: (0, 0, 0)),
            pl.BlockSpec((_V, 1, _D), lambda gi: (0, 0, 0)),
            pl.BlockSpec((1, _BM2, _FT), nei_idx),
            pl.BlockSpec((_V, _FT, _D), lambda gi: (0, 0, 0)),
            pl.BlockSpec((_V, 1, _FT), lambda gi: (0, 0, 0)),
            pl.BlockSpec((_V, _FT, _FT), lambda gi: (0, 0, 0)),
            pl.BlockSpec((_V, 1, _FT), lambda gi: (0, 0, 0)),
        ],
        out_specs=pl.BlockSpec(memory_space=pltpu.SMEM),
        out_shape=jax.ShapeDtypeStruct((1, 1), jnp.float32),
        scratch_shapes=[
            pltpu.VMEM((_V, _N, _D), jnp.float32),
            pltpu.VMEM((_V, _N, _D), jnp.float32),
            pltpu.VMEM((_V, _D, _D), jnp.float32),
            pltpu.VMEM((_D, _D), jnp.float32),
            pltpu.VMEM((_V, 1, _D), jnp.float32),
            pltpu.VMEM((_V, 1, _D), jnp.float32),
            pltpu.VMEM((_V, 1, _D), jnp.float32),
            pltpu.SMEM((1, 1), jnp.float32),
            pltpu.SMEM((1, 1), jnp.float32),
            pltpu.SMEM((1, 1), jnp.float32),
        ],
        compiler_params=pltpu.CompilerParams(
            dimension_semantics=("arbitrary",),
        ),
    )(adj, x, ew1, eb1.reshape(_V, 1, _H), ew2, eb2.reshape(_V, 1, _D), w0,
      g.reshape(_V, 1, _D), bb.reshape(_V, 1, _D), nei_sum, dw1,
      db1.reshape(_V, 1, _FT), dw2, db2.reshape(_V, 1, _FT))


def kernel(x, adj, enc_w1, enc_b1, enc_w2, enc_b2, dec_l0_w, dec_l0_b, bn_g,
           bn_b, dec_l1_w, dec_l1_b, dec_l2_w, dec_l2_b, idx_p, epoch):
    slots = (epoch + (_NEI // _SAMP) * jnp.arange(_SLOTS, dtype=jnp.int32)) % _NEI
    sel = jnp.take(idx_p, slots, axis=1)
    nei_sum = _nei_sum_sc(x, sel)
    out = _mega(adj, x, enc_w1, enc_b1, enc_w2, enc_b2, dec_l0_w, bn_g, bn_b,
                nei_sum, dec_l1_w, dec_l1_b, dec_l2_w, dec_l2_b)
    return out[0, 0]


# speedup vs baseline: 1.0491x; 1.0491x over previous
import jax
import jax.numpy as jnp
from jax import lax
from jax.experimental import pallas as pl
from jax.experimental.pallas import tpu as pltpu
from jax.experimental.pallas import tpu_sc as plsc

_V = 2
_N = 8192
_FT = 128
_H = 128
_D = 64
_NEI = 10
_SAMP = 5
_ALPHA = 0.5
_BETA = 1.0
_SLOTS = _SAMP + 1

_NW = 32
_RPW = (_V * _N) // _NW


_CHUNK = 128
_NCH = _RPW // _CHUNK


def _sc_nei_body(x_hbm, sel_hbm, out_hbm, idx_v, rows_v, sem):
    wid = lax.axis_index("s") * 2 + lax.axis_index("c")
    base = wid * _RPW
    view = base // _N
    c0 = (wid % 16) * _NCH
    pltpu.sync_copy(sel_hbm.at[view, :, pl.ds(c0, _NCH), :], idx_v)
    first = [
        pltpu.async_copy(x_hbm.at[idx_v.at[0, c]],
                         rows_v.at[pl.ds(c * _CHUNK, _CHUNK), :], sem)
        for c in range(_NCH)
    ]
    for d in first:
        d.wait()
    adds = [
        pltpu.async_copy(x_hbm.at[idx_v.at[j, c]],
                         rows_v.at[pl.ds(c * _CHUNK, _CHUNK), :], sem, add=True)
        for j in range(1, _SLOTS)
        for c in range(_NCH)
    ]
    for d in adds:
        d.wait()
    pltpu.sync_copy(rows_v, out_hbm.at[pl.ds(base, _RPW)])


def _nei_sum_sc(x, sel):
    k = pl.kernel(
        _sc_nei_body,
        out_type=jax.ShapeDtypeStruct((_V * _N, _FT), jnp.float32),
        mesh=plsc.VectorSubcoreMesh(core_axis_name="c", subcore_axis_name="s"),
        scratch_types=[
            pltpu.VMEM((_SLOTS, _NCH, _CHUNK), jnp.int32),
            pltpu.VMEM((_RPW, _FT), jnp.float32),
            pltpu.SemaphoreType.DMA,
        ],
    )
    return k(x, sel.reshape(_V, _SLOTS, _N // _CHUNK, _CHUNK)).reshape(_V, _N, _FT)


_BM = 512
_NB = _N // _BM
_NSP = _NB * _V
_BM2 = 1024
_ND = _N // _BM2
_NT = _NSP + _ND


def _mega_body(adj_ref, x_ref, ew1_ref, eb1_ref, ew2_ref, eb2_ref, w0_ref,
               g_ref, bb_ref, nei_ref, dw1_ref, db1_ref, dw2_ref, db2_ref,
               out_ref, ha_v, hp_v, gram_v, ic_v, sum_v, a_v, be_v,
               misc_s, e1_s, e2_s):
    g = pl.program_id(0)
    eye = (lax.broadcasted_iota(jnp.int32, (_D, _D), 0)
           == lax.broadcasted_iota(jnp.int32, (_D, _D), 1)).astype(jnp.float32)
    @pl.when(g < _NSP)
    def _spmm_phase():
        i = g // _NB
        b = g % _NB

        @pl.when(b == 0)
        def _():
            h = lax.dot_general(x_ref[...], ew1_ref[i], (((1,), (1,)), ((), ())),
                                preferred_element_type=jnp.float32)
            h = jnp.maximum(h + eb1_ref[i, 0:1, :], 0.0)
            ha = lax.dot_general(h, ew2_ref[i], (((1,), (1,)), ((), ())),
                                 preferred_element_type=jnp.float32)
            ha_v[i] = ha + eb2_ref[i, 0:1, :]
            gram_v[i] = jnp.zeros((_D, _D), jnp.float32)
            sum_v[i] = jnp.zeros((1, _D), jnp.float32)

            @pl.when(i == 0)
            def _():
                e1_s[0, 0] = 0.0
                e2_s[0, 0] = 0.0

            @pl.when(i == 1)
            def _():
                ic_v[...] = jnp.zeros((_D, _D), jnp.float32)

        hp = lax.dot_general(adj_ref[0], ha_v[i], (((1,), (0,)), ((), ())),
                             preferred_element_type=jnp.float32)
        hp_v[i, pl.ds(b * _BM, _BM)] = hp
        gram_v[i] += lax.dot_general(hp, hp, (((0,), (0,)), ((), ())),
                                     preferred_element_type=jnp.float32)
        sum_v[i] += jnp.sum(hp, axis=0, keepdims=True)

        @pl.when(i == 1)
        def _():
            ic_v[...] += lax.dot_general(hp, hp_v[0, pl.ds(b * _BM, _BM)],
                                         (((0,), (0,)), ((), ())),
                                         preferred_element_type=jnp.float32)

        @pl.when(b == _NB - 1)
        def _():
            gram = gram_v[i]
            rn = jnp.sqrt(jnp.sum(gram * gram, axis=1, keepdims=True))
            c = gram / jnp.maximum(rn, 1e-12)
            d = jnp.sum(c * eye, axis=1, keepdims=True)
            intra = jnp.sum((d - 1.0) ** 2) + (jnp.sum(c * c) - jnp.sum(d * d))

            @pl.when(i == 0)
            def _():
                misc_s[0, 0] = _ALPHA * intra

            s = sum_v[i] * (1.0 / _N)
            w0 = w0_ref[i]
            mean_u = lax.dot_general(s, w0, (((1,), (1,)), ((), ())),
                                     preferred_element_type=jnp.float32)
            m1 = lax.dot_general(w0, gram, (((1,), (0,)), ((), ())),
                                 preferred_element_type=jnp.float32)
            m2 = lax.dot_general(m1, w0, (((1,), (1,)), ((), ())),
                                 preferred_element_type=jnp.float32)
            ediag = jnp.sum(m2 * eye, axis=0, keepdims=True) * (1.0 / _N)
            var = ediag - mean_u * mean_u
            inv_std = lax.rsqrt(var + 1e-5)
            a = g_ref[i, 0:1, :] * inv_std
            a_v[i] = a
            be_v[i] = bb_ref[i, 0:1, :] - mean_u * a

            @pl.when(i == 1)
            def _():
                ic = ic_v[...]
                rn2 = jnp.sqrt(jnp.sum(ic * ic, axis=1, keepdims=True))
                icn = ic / jnp.maximum(rn2, 1e-12)
                misc_s[0, 0] += _ALPHA * intra - jnp.sum(icn * eye)

    @pl.when(((g >= _NB) & (g < _NB + _ND)) | (g >= _NSP))
    def _decoder_phase():
        i = jnp.where(g >= _NSP, 1, 0)
        bb = jnp.where(g >= _NSP, g - _NSP, g - _NB)

        hp = hp_v[i, pl.ds(bb * _BM2, _BM2)]
        u = lax.dot_general(hp, w0_ref[i], (((1,), (1,)), ((), ())),
                            preferred_element_type=jnp.float32)
        t = jnp.maximum(u * a_v[i] + be_v[i], 0.0)
        t2 = lax.dot_general(t, dw1_ref[i], (((1,), (1,)), ((), ())),
                             preferred_element_type=jnp.float32) + db1_ref[i]
        r = lax.dot_general(jnp.maximum(t2, 0.0), dw2_ref[i],
                            (((1,), (1,)), ((), ())),
                            preferred_element_type=jnp.float32) + db2_ref[i]
        d1 = r - x_ref[pl.ds(bb * _BM2, _BM2)]
        d2 = r - nei_ref[0] * (1.0 / _SAMP)
        e1_s[0, 0] += jnp.sum(d1 * d1)
        e2_s[0, 0] += jnp.sum(d2 * d2)

        @pl.when(g == _NT - 1)
        def _():
            out_ref[0, 0] = (misc_s[0, 0]
                             + _BETA * ((e1_s[0, 0] + e2_s[0, 0]) * (1.0 / _N)))


def _mega(adj, x, ew1, eb1, ew2, eb2, w0, g, bb, nei_sum, dw1, db1, dw2, db2):
    def adj_idx(gi):
        sp = gi < _NSP
        return (jnp.where(sp, gi // _NB, _V - 1),
                jnp.where(sp, gi % _NB, _NB - 1), 0)

    def nei_idx(gi):
        in0 = (gi >= _NB) & (gi < _NB + _ND)
        b = jnp.where(in0, gi - _NB,
                      jnp.where(gi >= _NSP, gi - _NSP,
                                jnp.where(gi < _NB, 0, _ND - 1)))
        return (jnp.where(gi >= _NSP, 1, 0), b, 0)

    return pl.pallas_call(
        _mega_body,
        grid=(_NT,),
        in_specs=[
            pl.BlockSpec((1, _BM, _N), adj_idx),
            pl.BlockSpec((_N, _FT), lambda gi: (0, 0)),
            pl.BlockSpec((_V, _H, _FT), lambda gi: (0, 0, 0)),
            pl.BlockSpec((_V, 1, _H), lambda gi: (0, 0, 0)),
            pl.BlockSpec((_V, _D, _H), lambda gi: (0, 0, 0)),
            pl.BlockSpec((_V, 1, _D), lambda gi: (0, 0, 0)),
            pl.BlockSpec((_V, _D, _D), lambda gi: (0, 0, 0)),
            pl.BlockSpec((_V, 1, _D), lambda gi: (0, 0, 0)),
            pl.BlockSpec((_V, 1, _D), lambda gi: (0, 0, 0)),
            pl.BlockSpec((1, _BM2, _FT), nei_idx),
            pl.BlockSpec((_V, _FT, _D), lambda gi: (0, 0, 0)),
            pl.BlockSpec((_V, 1, _FT), lambda gi: (0, 0, 0)),
            pl.BlockSpec((_V, _FT, _FT), lambda gi: (0, 0, 0)),
            pl.BlockSpec((_V, 1, _FT), lambda gi: (0, 0, 0)),
        ],
        out_specs=pl.BlockSpec(memory_space=pltpu.SMEM),
        out_shape=jax.ShapeDtypeStruct((1, 1), jnp.float32),
        scratch_shapes=[
            pltpu.VMEM((_V, _N, _D), jnp.float32),
            pltpu.VMEM((_V, _N, _D), jnp.float32),
            pltpu.VMEM((_V, _D, _D), jnp.float32),
            pltpu.VMEM((_D, _D), jnp.float32),
            pltpu.VMEM((_V, 1, _D), jnp.float32),
            pltpu.VMEM((_V, 1, _D), jnp.float32),
            pltpu.VMEM((_V, 1, _D), jnp.float32),
            pltpu.SMEM((1, 1), jnp.float32),
            pltpu.SMEM((1, 1), jnp.float32),
            pltpu.SMEM((1, 1), jnp.float32),
        ],
        compiler_params=pltpu.CompilerParams(
            dimension_semantics=("arbitrary",),
        ),
    )(adj, x, ew1, eb1.reshape(_V, 1, _H), ew2, eb2.reshape(_V, 1, _D), w0,
      g.reshape(_V, 1, _D), bb.reshape(_V, 1, _D), nei_sum, dw1,
      db1.reshape(_V, 1, _FT), dw2, db2.reshape(_V, 1, _FT))


def kernel(x, adj, enc_w1, enc_b1, enc_w2, enc_b2, dec_l0_w, dec_l0_b, bn_g,
           bn_b, dec_l1_w, dec_l1_b, dec_l2_w, dec_l2_b, idx_p, epoch):
    slots = (epoch + (_NEI // _SAMP) * jnp.arange(_SLOTS, dtype=jnp.int32)) % _NEI
    sel = jnp.take(idx_p, slots, axis=1)
    nei_sum = _nei_sum_sc(x, sel)
    out = _mega(adj, x, enc_w1, enc_b1, enc_w2, enc_b2, dec_l0_w, bn_g, bn_b,
                nei_sum, dec_l1_w, dec_l1_b, dec_l2_w, dec_l2_b)
    return out[0, 0]
